# baseline (speedup 1.0000x reference)
import functools

import jax
import jax.numpy as jnp
from jax import lax
from jax.experimental import pallas as pl
from jax.experimental.pallas import tpu as pltpu
from jax.experimental.pallas import tpu_sc as plsc

R = 16
INT = 8
WREP = 49.0
HW = 64
NPIX = HW * HW
C = 128
HID = 512
NB = 4
NT = 3
BATCH = 2
ROWS = BATCH * R * R
KEYS = R * R
NPIXB = BATCH * NPIX
NORM = float(BATCH * NPIX)
NW = 32
BPW = NPIXB // NW
CHUNK = 128


def _shift_rows(x, off):
    if off == 0:
        return x
    z = jnp.zeros((abs(off), x.shape[1]), x.dtype)
    if off > 0:
        return jnp.concatenate([x[off:], z], axis=0)
    return jnp.concatenate([z, x[:off]], axis=0)


def _shift4(x, d, axis):
    if d == 0:
        return x
    n = x.shape[axis]
    zshape = list(x.shape)
    zshape[axis] = abs(d)
    z = jnp.zeros(zshape, x.dtype)
    if d > 0:
        return jnp.concatenate([lax.slice_in_dim(x, d, n, axis=axis), z],
                               axis=axis)
    return jnp.concatenate([z, lax.slice_in_dim(x, 0, n + d, axis=axis)],
                           axis=axis)


def _towers_body(temb_ref, fcw_ref, fcb_ref, w1_ref, b1_ref, dw_ref,
                 dwb_ref, w2_ref, b2_ref, bng_ref, bnb_ref):
    s = lax.broadcasted_iota(jnp.int32, (ROWS, 1), 0)
    b_id = s // (R * R)
    sl = lax.broadcasted_iota(jnp.int32, (1, ROWS), 1)
    hl = (sl // R) % R
    wl = sl % R
    wt_l = (jnp.where(hl == INT, WREP, 1.0)
            * jnp.where(wl == INT, WREP, 1.0))
    hiprec = jax.lax.Precision.HIGHEST

    prod = None
    for t in range(NT):
        e = jnp.mean(temb_ref[t], axis=1)
        x0 = jax.nn.relu(
            lax.dot_general(e, fcw_ref[t], (((1,), (1,)), ((), ())),
                            preferred_element_type=jnp.float32)
            + fcb_ref[t:t + 1])
        x = jnp.where(b_id == 0, x0[0:1], x0[1:2])

        for k in range(NB):
            mu = lax.dot_general(wt_l, x, (((1,), (0,)), ((), ())),
                                 preferred_element_type=jnp.float32,
                                 precision=hiprec) / NORM
            ex2 = lax.dot_general(wt_l, x * x, (((1,), (0,)), ((), ())),
                                  preferred_element_type=jnp.float32,
                                  precision=hiprec) / NORM
            var = ex2 - mu * mu
            xn = (x - mu) / jnp.sqrt(var + 1e-5)
            xn = xn * bng_ref[t, k:k + 1] + bnb_ref[t, k:k + 1]
            h = jax.nn.relu(
                lax.dot_general(xn, w1_ref[t, k], (((1,), (1,)), ((), ())),
                                preferred_element_type=jnp.float32)
                + b1_ref[t, k:k + 1])
            h4 = h.reshape(BATCH, R, R, HID)
            h4 = jnp.concatenate([h4[:, :, R - 1:, :], h4[:, :, :R - 1, :]],
                                 axis=2)
            acc = None
            for ky in range(3):
                hy = _shift4(h4, ky - 1, 1)
                for kx in range(3):
                    kv = dw_ref[t, k, 3 * ky + kx:3 * ky + kx + 1]
                    term = _shift4(hy, kx - 1, 2) * kv
                    acc = term if acc is None else acc + term
            h = jax.nn.relu(acc + dwb_ref[t, k:k + 1]).reshape(ROWS, HID)
            x = (x
                 + lax.dot_general(h, w2_ref[t, k], (((1,), (1,)), ((), ())),
                                   preferred_element_type=jnp.float32)
                 + b2_ref[t, k:k + 1])
        prod = x if prod is None else prod * x

    return prod / (jnp.sqrt(jnp.sum(prod * prod, axis=1,
                                    keepdims=True)) + 1e-6)


def _front_kernel(v_ref, temb_ref, fcw_ref, fcb_ref, w1_ref, b1_ref, dw_ref,
                  dwb_ref, w2_ref, b2_ref, bng_ref, bnb_ref,
                  keys_ref, idx_ref):
    keys = _towers_body(temb_ref, fcw_ref, fcb_ref, w1_ref, b1_ref, dw_ref,
                        dwb_ref, w2_ref, b2_ref, bng_ref, bnb_ref)
    keys_ref[...] = keys

    vf = v_ref[...]
    pn = vf / (jnp.sqrt(jnp.sum(vf * vf, axis=1, keepdims=True)) + 1e-6)
    pn_aug = jnp.concatenate(
        [pn * -2.0, jnp.ones((NPIXB, 1), jnp.float32)], axis=1)
    kn2 = jnp.sum(keys * keys, axis=1, keepdims=True)
    keys_aug = jnp.concatenate([keys, kn2], axis=1)
    for b in range(BATCH):
        ka = keys_aug[b * KEYS:(b + 1) * KEYS]
        d2 = lax.dot_general(pn_aug[b * NPIX:(b + 1) * NPIX], ka,
                             (((1,), (1,)), ((), ())),
                             preferred_element_type=jnp.float32)
        m = jnp.min(d2, axis=1, keepdims=True)
        ji = lax.broadcasted_iota(jnp.int32, d2.shape, 1)
        idx = jnp.min(jnp.where(d2 == m, ji, KEYS), axis=1, keepdims=True)
        idx_ref[b * NPIX:(b + 1) * NPIX] = idx + b * KEYS


_sc_mesh = plsc.VectorSubcoreMesh(core_axis_name="c", subcore_axis_name="s")


@functools.partial(
    pl.kernel, mesh=_sc_mesh,
    out_type=jax.ShapeDtypeStruct((NPIXB, C), jnp.float32),
    scratch_types=[
        pltpu.VMEM((BPW // CHUNK, CHUNK), jnp.int32),
        pltpu.VMEM((BPW, C), jnp.float32),
        pltpu.SemaphoreType.DMA,
    ],
)
def _sc_gather(keys_hbm, idx_hbm, out_hbm, idx_v, rows_v, sem):
    wid = lax.axis_index("s") * 2 + lax.axis_index("c")
    pltpu.sync_copy(idx_hbm.at[wid], idx_v)
    for j in range(BPW // CHUNK):
        pltpu.async_copy(keys_hbm.at[idx_v.at[j]],
                         rows_v.at[pl.ds(j * CHUNK, CHUNK)], sem).wait()
    pltpu.sync_copy(rows_v, out_hbm.at[pl.ds(wid * BPW, BPW)])


def _back_kernel(v_ref, t_ref, tvw1_ref, tvb1_ref, tvw2_ref, tvb2_ref,
                 ttw1_ref, ttb1_ref, ttw2_ref, ttb2_ref, o_ref):
    vf = v_ref[...]
    tf = t_ref[...]
    s = lax.broadcasted_iota(jnp.int32, (NPIXB, 1), 0)
    hpos = (s // HW) % HW
    wpos = s % HW

    def blend_coef(vcur, stride, pos):
        tprev = _shift_rows(tf, -stride)
        num = jnp.sum(vcur * tprev, axis=1, keepdims=True)
        den = jnp.maximum(
            jnp.sqrt(jnp.sum(vcur * vcur, axis=1, keepdims=True))
            * jnp.sqrt(jnp.sum(tprev * tprev, axis=1, keepdims=True)), 1e-8)
        return jnp.where(pos == 0, 0.0, jnp.exp(-(1.0 - num / den)))

    def linscan(vcur, stride, pos):
        A = blend_coef(vcur, stride, pos)
        Bv = (1.0 - A) * vcur
        k = 1
        while k < HW:
            live = pos >= k
            Am = jnp.where(live, A, 0.0)
            Bv = Am * _shift_rows(Bv, -k * stride) + Bv
            A = A * jnp.where(live, _shift_rows(A, -k * stride), 1.0)
            k *= 2
        return Bv

    vr = linscan(vf, 1, wpos)
    vc = linscan(vr, HW, hpos)

    def cosd(a, b):
        num = jnp.sum(a * b, axis=1, keepdims=True)
        den = jnp.maximum(
            jnp.sqrt(jnp.sum(a * a, axis=1, keepdims=True))
            * jnp.sqrt(jnp.sum(b * b, axis=1, keepdims=True)), 1e-8)
        return 1.0 - num / den

    d_tv = cosd(vc, tf)
    tnext = _shift_rows(tf, 1)
    d_tt = jnp.where(s % NPIX == NPIX - 1, 0.0, cosd(tf, tnext))

    def mlp(d, w1, b1, w2, b2):
        h = jax.nn.relu(d * w1 + b1)
        return jnp.sum(h * w2, axis=1, keepdims=True) + b2

    gate = jax.nn.sigmoid(
        mlp(d_tv, tvw1_ref[...], tvb1_ref[...], tvw2_ref[...], tvb2_ref[...])
        + mlp(d_tt, ttw1_ref[...], ttb1_ref[...], ttw2_ref[...], ttb2_ref[...]))
    o_ref[...] = vc * gate


def kernel(V, tA, tB, tAB, params):
    towers = [params[n] for n in ('tA', 'tB', 'tAB')]
    temb = jnp.stack([tA, tB, tAB])
    fcw = jnp.stack([p['fc_w'] for p in towers])
    fcb = jnp.stack([p['fc_b'] for p in towers])

    def blk(name):
        return jnp.stack([jnp.stack([b[name] for b in p['blocks']])
                          for p in towers])

    w1, b1, dwb = blk('w1'), blk('b1'), blk('dwb')
    w2, b2 = blk('w2'), blk('b2')
    bng, bnb = blk('bn_g'), blk('bn_b')
    dw = blk('dw').reshape(NT, NB, HID, 9).transpose(0, 1, 3, 2)

    vn = jnp.transpose(V, (0, 2, 3, 1)).reshape(NPIXB, C)

    keys, idx = pl.pallas_call(
        _front_kernel,
        out_shape=(jax.ShapeDtypeStruct((ROWS, C), jnp.float32),
                   jax.ShapeDtypeStruct((NPIXB, 1), jnp.int32)),
    )(vn, temb, fcw, fcb, w1, b1, dw, dwb, w2, b2, bng, bnb)

    idx3 = idx.reshape(NW, BPW // CHUNK, CHUNK)
    tr = _sc_gather(keys, idx3)

    mlp_params = (
        params['tv']['w1'].reshape(1, 256), params['tv']['b1'].reshape(1, 256),
        params['tv']['w2'].reshape(1, 256), params['tv']['b2'].reshape(1, 1),
        params['tt']['w1'].reshape(1, 256), params['tt']['b1'].reshape(1, 256),
        params['tt']['w2'].reshape(1, 256), params['tt']['b2'].reshape(1, 1),
    )
    out = pl.pallas_call(
        _back_kernel,
        out_shape=jax.ShapeDtypeStruct((NPIXB, C), jnp.float32),
    )(vn, tr, *mlp_params)
    return jnp.transpose(out.reshape(BATCH, HW, HW, C), (0, 3, 1, 2))


# speedup vs baseline: 1.8213x; 1.8213x over previous
import jax
import jax.numpy as jnp
from jax import lax
from jax.experimental import pallas as pl

R = 10
HW = 64
NPIX = HW * HW
C = 128
HID = 512
NB = 4
NT = 3
BATCH = 2
ROWS = BATCH * R * R
KEYS = R * R
KEYSP = 104
NORM = float(BATCH * NPIX)


def _shift_rows(x, off):
    if off == 0:
        return x
    z = jnp.zeros((abs(off), x.shape[1]), x.dtype)
    if off > 0:
        return jnp.concatenate([x[off:], z], axis=0)
    return jnp.concatenate([z, x[:off]], axis=0)


def _shift4(x, d, axis):
    if d == 0:
        return x
    n = x.shape[axis]
    zshape = list(x.shape)
    zshape[axis] = abs(d)
    z = jnp.zeros(zshape, x.dtype)
    if d > 0:
        return jnp.concatenate([lax.slice_in_dim(x, d, n, axis=axis), z],
                               axis=axis)
    return jnp.concatenate([z, lax.slice_in_dim(x, 0, n + d, axis=axis)],
                           axis=axis)


def _towers_kernel(temb_ref, fcw_ref, fcb_ref, w1_ref, b1_ref, dw_ref,
                   dwb_ref, w2_ref, b2_ref, bng_ref, bnb_ref, keys_ref):
    s = lax.broadcasted_iota(jnp.int32, (ROWS, 1), 0)
    b_id = s // (R * R)
    sl = lax.broadcasted_iota(jnp.int32, (1, ROWS), 1)
    hl = (sl // R) % R
    wl = sl % R
    wt_l = (jnp.where((hl == 4) | (hl == 5), 28.0, 1.0)
            * jnp.where(wl == 7, 55.0, 1.0))
    hiprec = jax.lax.Precision.HIGHEST

    prod = None
    for t in range(NT):
        e = jnp.mean(temb_ref[t], axis=1)
        x0 = jax.nn.relu(
            lax.dot_general(e, fcw_ref[t], (((1,), (1,)), ((), ())),
                            preferred_element_type=jnp.float32)
            + fcb_ref[t:t + 1])
        x = jnp.where(b_id == 0, x0[0:1], x0[1:2])

        for k in range(NB):
            mu = lax.dot_general(wt_l, x, (((1,), (0,)), ((), ())),
                                 preferred_element_type=jnp.float32,
                                 precision=hiprec) / NORM
            ex2 = lax.dot_general(wt_l, x * x, (((1,), (0,)), ((), ())),
                                  preferred_element_type=jnp.float32,
                                  precision=hiprec) / NORM
            var = ex2 - mu * mu
            xn = (x - mu) / jnp.sqrt(var + 1e-5)
            xn = xn * bng_ref[t, k:k + 1] + bnb_ref[t, k:k + 1]
            h = jax.nn.relu(
                lax.dot_general(xn, w1_ref[t, k], (((1,), (1,)), ((), ())),
                                preferred_element_type=jnp.float32)
                + b1_ref[t, k:k + 1])
            h4 = h.reshape(BATCH, R, R, HID)
            h4 = jnp.concatenate([h4[:, :, R - 1:, :], h4[:, :, :R - 1, :]],
                                 axis=2)
            acc = None
            for ky in range(3):
                hy = _shift4(h4, ky - 1, 1)
                for kx in range(3):
                    kv = dw_ref[t, k, 3 * ky + kx:3 * ky + kx + 1]
                    term = _shift4(hy, kx - 1, 2) * kv
                    acc = term if acc is None else acc + term
            h = jax.nn.relu(acc + dwb_ref[t, k:k + 1]).reshape(ROWS, HID)
            x = (x
                 + lax.dot_general(h, w2_ref[t, k], (((1,), (1,)), ((), ())),
                                   preferred_element_type=jnp.float32)
                 + b2_ref[t, k:k + 1])
        prod = x if prod is None else prod * x

    keys_ref[...] = prod / (jnp.sqrt(jnp.sum(prod * prod, axis=1,
                                             keepdims=True)) + 1e-6)


def _towers(temb, fcw, fcb, w1, b1, dw, dwb, w2, b2, bng, bnb):
    return pl.pallas_call(
        _towers_kernel,
        out_shape=jax.ShapeDtypeStruct((ROWS, C), jnp.float32),
    )(temb, fcw, fcb, w1, b1, dw, dwb, w2, b2, bng, bnb)


def _knn_kernel(v_ref, k_ref, tr_ref):
    v = v_ref[0]
    keys = k_ref[0]
    pn = v / (jnp.sqrt(jnp.sum(v * v, axis=1, keepdims=True)) + 1e-6)
    kn2 = jnp.sum(keys * keys, axis=1, keepdims=True)
    keys_aug = jnp.concatenate([keys, kn2], axis=1)
    pn_aug = jnp.concatenate(
        [pn * -2.0, jnp.ones((pn.shape[0], 1), jnp.float32)], axis=1)
    d2 = lax.dot_general(pn_aug, keys_aug, (((1,), (1,)), ((), ())),
                         preferred_element_type=jnp.float32)
    ji = lax.broadcasted_iota(jnp.int32, d2.shape, 1)
    d2 = jnp.where(ji < KEYS, d2, 3.0e38)
    m = jnp.min(d2, axis=1, keepdims=True)
    idx = jnp.min(jnp.where(d2 == m, ji, KEYSP), axis=1, keepdims=True)
    onehot = (ji == idx).astype(jnp.float32)
    tr_ref[0] = lax.dot_general(onehot, keys, (((1,), (0,)), ((), ())),
                                preferred_element_type=jnp.float32)


def _knn(vn, keys):
    return pl.pallas_call(
        _knn_kernel,
        grid=(BATCH,),
        in_specs=[
            pl.BlockSpec((1, NPIX, C), lambda b: (b, 0, 0)),
            pl.BlockSpec((1, KEYSP, C), lambda b: (b, 0, 0)),
        ],
        out_specs=pl.BlockSpec((1, NPIX, C), lambda b: (b, 0, 0)),
        out_shape=jax.ShapeDtypeStruct((BATCH, NPIX, C), jnp.float32),
    )(vn, keys)


def _scanfin_kernel(v_ref, t_ref, tvw1_ref, tvb1_ref, tvw2_ref, tvb2_ref,
                    ttw1_ref, ttb1_ref, ttw2_ref, ttb2_ref, o_ref):
    vf = v_ref[0]
    tf = t_ref[0]
    s = lax.broadcasted_iota(jnp.int32, (NPIX, 1), 0)
    hpos = s // HW
    wpos = s % HW

    def blend_coef(vcur, stride, pos):
        tprev = _shift_rows(tf, -stride)
        num = jnp.sum(vcur * tprev, axis=1, keepdims=True)
        den = jnp.maximum(
            jnp.sqrt(jnp.sum(vcur * vcur, axis=1, keepdims=True))
            * jnp.sqrt(jnp.sum(tprev * tprev, axis=1, keepdims=True)), 1e-8)
        return jnp.where(pos == 0, 0.0, jnp.exp(-(1.0 - num / den)))

    def linscan(vcur, stride, pos):
        A = blend_coef(vcur, stride, pos)
        Bv = (1.0 - A) * vcur
        k = 1
        while k < HW:
            live = pos >= k
            Am = jnp.where(live, A, 0.0)
            Bv = Am * _shift_rows(Bv, -k * stride) + Bv
            A = A * jnp.where(live, _shift_rows(A, -k * stride), 1.0)
            k *= 2
        return Bv

    vr = linscan(vf, 1, wpos)
    vc = linscan(vr, HW, hpos)

    def cosd(a, b):
        num = jnp.sum(a * b, axis=1, keepdims=True)
        den = jnp.maximum(
            jnp.sqrt(jnp.sum(a * a, axis=1, keepdims=True))
            * jnp.sqrt(jnp.sum(b * b, axis=1, keepdims=True)), 1e-8)
        return 1.0 - num / den

    d_tv = cosd(vc, tf)
    tnext = _shift_rows(tf, 1)
    d_tt = jnp.where(s == NPIX - 1, 0.0, cosd(tf, tnext))

    def mlp(d, w1, b1, w2, b2):
        h = jax.nn.relu(d * w1 + b1)
        return jnp.sum(h * w2, axis=1, keepdims=True) + b2

    gate = jax.nn.sigmoid(
        mlp(d_tv, tvw1_ref[...], tvb1_ref[...], tvw2_ref[...], tvb2_ref[...])
        + mlp(d_tt, ttw1_ref[...], ttb1_ref[...], ttw2_ref[...], ttb2_ref[...]))
    o_ref[0] = vc * gate


def _scanfin(vn, tr, mlp_params):
    vec = lambda: pl.BlockSpec((1, 256), lambda b: (0, 0))
    scl = lambda: pl.BlockSpec((1, 1), lambda b: (0, 0))
    return pl.pallas_call(
        _scanfin_kernel,
        grid=(BATCH,),
        in_specs=[
            pl.BlockSpec((1, NPIX, C), lambda b: (b, 0, 0)),
            pl.BlockSpec((1, NPIX, C), lambda b: (b, 0, 0)),
            vec(), vec(), vec(), scl(), vec(), vec(), vec(), scl(),
        ],
        out_specs=pl.BlockSpec((1, NPIX, C), lambda b: (b, 0, 0)),
        out_shape=jax.ShapeDtypeStruct((BATCH, NPIX, C), jnp.float32),
    )(vn, tr, *mlp_params)


def kernel(V, tA, tB, tAB, params):
    towers = [params[n] for n in ('tA', 'tB', 'tAB')]
    temb = jnp.stack([tA, tB, tAB])
    fcw = jnp.stack([p['fc_w'] for p in towers])
    fcb = jnp.stack([p['fc_b'] for p in towers])

    def blk(name):
        return jnp.stack([jnp.stack([b[name] for b in p['blocks']])
                          for p in towers])

    w1, b1, dwb = blk('w1'), blk('b1'), blk('dwb')
    w2, b2 = blk('w2'), blk('b2')
    bng, bnb = blk('bn_g'), blk('bn_b')
    dw = blk('dw').reshape(NT, NB, HID, 9).transpose(0, 1, 3, 2)

    keys = _towers(temb, fcw, fcb, w1, b1, dw, dwb, w2, b2, bng, bnb)
    keys = jnp.pad(keys.reshape(BATCH, KEYS, C),
                   ((0, 0), (0, KEYSP - KEYS), (0, 0)))

    vn = jnp.transpose(V, (0, 2, 3, 1)).reshape(BATCH, NPIX, C)
    tr = _knn(vn, keys)

    mlp_params = (
        params['tv']['w1'].reshape(1, 256), params['tv']['b1'].reshape(1, 256),
        params['tv']['w2'].reshape(1, 256), params['tv']['b2'].reshape(1, 1),
        params['tt']['w1'].reshape(1, 256), params['tt']['b1'].reshape(1, 256),
        params['tt']['w2'].reshape(1, 256), params['tt']['b2'].reshape(1, 1),
    )
    out = _scanfin(vn, tr, mlp_params)
    return jnp.transpose(out.reshape(BATCH, HW, HW, C), (0, 3, 1, 2))


# speedup vs baseline: 1.8273x; 1.0033x over previous
import jax
import jax.numpy as jnp
from jax import lax
from jax.experimental import pallas as pl

R = 10
HW = 64
NPIX = HW * HW
C = 128
HID = 512
NB = 4
NT = 3
BATCH = 2
ROWS = BATCH * R * R
KEYS = R * R
KEYSP = 104
NORM = float(BATCH * NPIX)


def _shift_rows(x, off):
    if off == 0:
        return x
    z = jnp.zeros((abs(off), x.shape[1]), x.dtype)
    if off > 0:
        return jnp.concatenate([x[off:], z], axis=0)
    return jnp.concatenate([z, x[:off]], axis=0)


def _shift4(x, d, axis):
    if d == 0:
        return x
    n = x.shape[axis]
    zshape = list(x.shape)
    zshape[axis] = abs(d)
    z = jnp.zeros(zshape, x.dtype)
    if d > 0:
        return jnp.concatenate([lax.slice_in_dim(x, d, n, axis=axis), z],
                               axis=axis)
    return jnp.concatenate([z, lax.slice_in_dim(x, 0, n + d, axis=axis)],
                           axis=axis)


def _towers_kernel(temb_ref, fcw_ref, fcb_ref, w1_ref, b1_ref, dw_ref,
                   dwb_ref, w2_ref, b2_ref, bng_ref, bnb_ref, keys_ref):
    s = lax.broadcasted_iota(jnp.int32, (ROWS, 1), 0)
    b_id = s // (R * R)
    hpos = (s // R) % R
    wpos = s % R
    sl = lax.broadcasted_iota(jnp.int32, (1, ROWS), 1)
    hl = (sl // R) % R
    wl = sl % R
    wt_l = (jnp.where((hl == 4) | (hl == 5), 28.0, 1.0)
            * jnp.where(wl == 7, 55.0, 1.0))
    hiprec = jax.lax.Precision.HIGHEST

    prod = None
    for t in range(NT):
        e = jnp.mean(temb_ref[t], axis=1)
        x0 = jax.nn.relu(
            lax.dot_general(e, fcw_ref[t], (((1,), (1,)), ((), ())),
                            preferred_element_type=jnp.float32)
            + fcb_ref[t:t + 1])
        x = jnp.where(b_id == 0, x0[0:1], x0[1:2])

        for k in range(NB):
            mu = lax.dot_general(wt_l, x, (((1,), (0,)), ((), ())),
                                 preferred_element_type=jnp.float32,
                                 precision=hiprec) / NORM
            ex2 = lax.dot_general(wt_l, x * x, (((1,), (0,)), ((), ())),
                                  preferred_element_type=jnp.float32,
                                  precision=hiprec) / NORM
            var = ex2 - mu * mu
            xn = (x - mu) / jnp.sqrt(var + 1e-5)
            xn = xn * bng_ref[t, k:k + 1] + bnb_ref[t, k:k + 1]
            h = jax.nn.relu(
                lax.dot_general(xn, w1_ref[t, k], (((1,), (1,)), ((), ())),
                                preferred_element_type=jnp.float32)
                + b1_ref[t, k:k + 1])
            h = jnp.where(wpos == 0, _shift_rows(h, R - 1), _shift_rows(h, -1))
            acc = None
            for ky in range(3):
                for kx in range(3):
                    dy, dx = ky - 1, kx - 1
                    m = (((hpos + dy) >= 0) & ((hpos + dy) < R)
                         & ((wpos + dx) >= 0) & ((wpos + dx) < R)
                         ).astype(h.dtype)
                    kv = dw_ref[t, k, 3 * ky + kx:3 * ky + kx + 1]
                    term = _shift_rows(h, dy * R + dx) * m * kv
                    acc = term if acc is None else acc + term
            h = jax.nn.relu(acc + dwb_ref[t, k:k + 1])
            x = (x
                 + lax.dot_general(h, w2_ref[t, k], (((1,), (1,)), ((), ())),
                                   preferred_element_type=jnp.float32)
                 + b2_ref[t, k:k + 1])
        prod = x if prod is None else prod * x

    keys_ref[...] = prod / (jnp.sqrt(jnp.sum(prod * prod, axis=1,
                                             keepdims=True)) + 1e-6)


def _towers(temb, fcw, fcb, w1, b1, dw, dwb, w2, b2, bng, bnb):
    return pl.pallas_call(
        _towers_kernel,
        out_shape=jax.ShapeDtypeStruct((ROWS, C), jnp.float32),
    )(temb, fcw, fcb, w1, b1, dw, dwb, w2, b2, bng, bnb)


def _knn_kernel(v_ref, k_ref, tr_ref):
    v = v_ref[0]
    keys = k_ref[0]
    pn = v / (jnp.sqrt(jnp.sum(v * v, axis=1, keepdims=True)) + 1e-6)
    kn2 = jnp.sum(keys * keys, axis=1, keepdims=True)
    keys_aug = jnp.concatenate([keys, kn2], axis=1)
    pn_aug = jnp.concatenate(
        [pn * -2.0, jnp.ones((pn.shape[0], 1), jnp.float32)], axis=1)
    d2 = lax.dot_general(pn_aug, keys_aug, (((1,), (1,)), ((), ())),
                         preferred_element_type=jnp.float32)
    ji = lax.broadcasted_iota(jnp.int32, d2.shape, 1)
    d2 = jnp.where(ji < KEYS, d2, 3.0e38)
    m = jnp.min(d2, axis=1, keepdims=True)
    idx = jnp.min(jnp.where(d2 == m, ji, KEYSP), axis=1, keepdims=True)
    onehot = (ji == idx).astype(jnp.float32)
    tr_ref[0] = lax.dot_general(onehot, keys, (((1,), (0,)), ((), ())),
                                preferred_element_type=jnp.float32)


def _knn(vn, keys):
    return pl.pallas_call(
        _knn_kernel,
        grid=(BATCH,),
        in_specs=[
            pl.BlockSpec((1, NPIX, C), lambda b: (b, 0, 0)),
            pl.BlockSpec((1, KEYSP, C), lambda b: (b, 0, 0)),
        ],
        out_specs=pl.BlockSpec((1, NPIX, C), lambda b: (b, 0, 0)),
        out_shape=jax.ShapeDtypeStruct((BATCH, NPIX, C), jnp.float32),
    )(vn, keys)


def _scanfin_kernel(v_ref, t_ref, tvw1_ref, tvb1_ref, tvw2_ref, tvb2_ref,
                    ttw1_ref, ttb1_ref, ttw2_ref, ttb2_ref, o_ref):
    vf = v_ref[0]
    tf = t_ref[0]
    s = lax.broadcasted_iota(jnp.int32, (NPIX, 1), 0)
    hpos = s // HW
    wpos = s % HW

    def blend_coef(vcur, stride, pos):
        tprev = _shift_rows(tf, -stride)
        num = jnp.sum(vcur * tprev, axis=1, keepdims=True)
        den = jnp.maximum(
            jnp.sqrt(jnp.sum(vcur * vcur, axis=1, keepdims=True))
            * jnp.sqrt(jnp.sum(tprev * tprev, axis=1, keepdims=True)), 1e-8)
        return jnp.where(pos == 0, 0.0, jnp.exp(-(1.0 - num / den)))

    def linscan(vcur, stride, pos):
        A = blend_coef(vcur, stride, pos)
        Bv = (1.0 - A) * vcur
        k = 1
        while k < HW:
            live = pos >= k
            Am = jnp.where(live, A, 0.0)
            Bv = Am * _shift_rows(Bv, -k * stride) + Bv
            A = A * jnp.where(live, _shift_rows(A, -k * stride), 1.0)
            k *= 2
        return Bv

    vr = linscan(vf, 1, wpos)
    vc = linscan(vr, HW, hpos)

    def cosd(a, b):
        num = jnp.sum(a * b, axis=1, keepdims=True)
        den = jnp.maximum(
            jnp.sqrt(jnp.sum(a * a, axis=1, keepdims=True))
            * jnp.sqrt(jnp.sum(b * b, axis=1, keepdims=True)), 1e-8)
        return 1.0 - num / den

    d_tv = cosd(vc, tf)
    tnext = _shift_rows(tf, 1)
    d_tt = jnp.where(s == NPIX - 1, 0.0, cosd(tf, tnext))

    def mlp(d, w1, b1, w2, b2):
        h = jax.nn.relu(d * w1 + b1)
        return jnp.sum(h * w2, axis=1, keepdims=True) + b2

    gate = jax.nn.sigmoid(
        mlp(d_tv, tvw1_ref[...], tvb1_ref[...], tvw2_ref[...], tvb2_ref[...])
        + mlp(d_tt, ttw1_ref[...], ttb1_ref[...], ttw2_ref[...], ttb2_ref[...]))
    o_ref[0] = vc * gate


def _scanfin(vn, tr, mlp_params):
    vec = lambda: pl.BlockSpec((1, 256), lambda b: (0, 0))
    scl = lambda: pl.BlockSpec((1, 1), lambda b: (0, 0))
    return pl.pallas_call(
        _scanfin_kernel,
        grid=(BATCH,),
        in_specs=[
            pl.BlockSpec((1, NPIX, C), lambda b: (b, 0, 0)),
            pl.BlockSpec((1, NPIX, C), lambda b: (b, 0, 0)),
            vec(), vec(), vec(), scl(), vec(), vec(), vec(), scl(),
        ],
        out_specs=pl.BlockSpec((1, NPIX, C), lambda b: (b, 0, 0)),
        out_shape=jax.ShapeDtypeStruct((BATCH, NPIX, C), jnp.float32),
    )(vn, tr, *mlp_params)


def kernel(V, tA, tB, tAB, params):
    towers = [params[n] for n in ('tA', 'tB', 'tAB')]
    temb = jnp.stack([tA, tB, tAB])
    fcw = jnp.stack([p['fc_w'] for p in towers])
    fcb = jnp.stack([p['fc_b'] for p in towers])

    def blk(name):
        return jnp.stack([jnp.stack([b[name] for b in p['blocks']])
                          for p in towers])

    w1, b1, dwb = blk('w1'), blk('b1'), blk('dwb')
    w2, b2 = blk('w2'), blk('b2')
    bng, bnb = blk('bn_g'), blk('bn_b')
    dw = blk('dw').reshape(NT, NB, HID, 9).transpose(0, 1, 3, 2)

    keys = _towers(temb, fcw, fcb, w1, b1, dw, dwb, w2, b2, bng, bnb)
    keys = jnp.pad(keys.reshape(BATCH, KEYS, C),
                   ((0, 0), (0, KEYSP - KEYS), (0, 0)))

    vn = jnp.transpose(V, (0, 2, 3, 1)).reshape(BATCH, NPIX, C)
    tr = _knn(vn, keys)

    mlp_params = (
        params['tv']['w1'].reshape(1, 256), params['tv']['b1'].reshape(1, 256),
        params['tv']['w2'].reshape(1, 256), params['tv']['b2'].reshape(1, 1),
        params['tt']['w1'].reshape(1, 256), params['tt']['b1'].reshape(1, 256),
        params['tt']['w2'].reshape(1, 256), params['tt']['b2'].reshape(1, 1),
    )
    out = _scanfin(vn, tr, mlp_params)
    return jnp.transpose(out.reshape(BATCH, HW, HW, C), (0, 3, 1, 2))


# speedup vs baseline: 1.8288x; 1.0008x over previous
import jax
import jax.numpy as jnp
from jax import lax
from jax.experimental import pallas as pl

R = 10
HW = 64
NPIX = HW * HW
C = 128
HID = 512
NB = 4
NT = 3
BATCH = 2
ROWS = BATCH * R * R
KEYS = R * R
KEYSP = 104
NORM = float(BATCH * NPIX)


def _shift_rows(x, off):
    if off == 0:
        return x
    z = jnp.zeros((abs(off), x.shape[1]), x.dtype)
    if off > 0:
        return jnp.concatenate([x[off:], z], axis=0)
    return jnp.concatenate([z, x[:off]], axis=0)


def _towers_kernel(temb_ref, fcw_ref, fcb_ref, w1_ref, b1_ref, dw_ref,
                   dwb_ref, w2_ref, b2_ref, bng_ref, bnb_ref, keys_ref):
    s = lax.broadcasted_iota(jnp.int32, (ROWS, 1), 0)
    b_id = s // (R * R)
    hpos = (s // R) % R
    wpos = s % R
    sl = lax.broadcasted_iota(jnp.int32, (1, ROWS), 1)
    hl = (sl // R) % R
    wl = sl % R
    wt_l = (jnp.where((hl == 4) | (hl == 5), 28.0, 1.0)
            * jnp.where(wl == 7, 55.0, 1.0))
    hiprec = jax.lax.Precision.HIGHEST

    prod = None
    for t in range(NT):
        e = jnp.mean(temb_ref[t], axis=1)
        x0 = jax.nn.relu(
            lax.dot_general(e, fcw_ref[t], (((1,), (1,)), ((), ())),
                            preferred_element_type=jnp.float32)
            + fcb_ref[t:t + 1])
        x = jnp.where(b_id == 0, x0[0:1], x0[1:2])

        for k in range(NB):
            mu = lax.dot_general(wt_l, x, (((1,), (0,)), ((), ())),
                                 preferred_element_type=jnp.float32,
                                 precision=hiprec) / NORM
            ex2 = lax.dot_general(wt_l, x * x, (((1,), (0,)), ((), ())),
                                  preferred_element_type=jnp.float32,
                                  precision=hiprec) / NORM
            var = ex2 - mu * mu
            xn = (x - mu) / jnp.sqrt(var + 1e-5)
            xn = xn * bng_ref[t, k:k + 1] + bnb_ref[t, k:k + 1]
            h = jax.nn.relu(
                lax.dot_general(xn, w1_ref[t, k], (((1,), (1,)), ((), ())),
                                preferred_element_type=jnp.float32)
                + b1_ref[t, k:k + 1])
            h = jnp.where(wpos == 0, _shift_rows(h, R - 1), _shift_rows(h, -1))
            acc = None
            for ky in range(3):
                for kx in range(3):
                    dy, dx = ky - 1, kx - 1
                    m = (((hpos + dy) >= 0) & ((hpos + dy) < R)
                         & ((wpos + dx) >= 0) & ((wpos + dx) < R)
                         ).astype(h.dtype)
                    kv = dw_ref[t, k, 3 * ky + kx:3 * ky + kx + 1]
                    term = _shift_rows(h, dy * R + dx) * m * kv
                    acc = term if acc is None else acc + term
            h = jax.nn.relu(acc + dwb_ref[t, k:k + 1])
            x = (x
                 + lax.dot_general(h, w2_ref[t, k], (((1,), (1,)), ((), ())),
                                   preferred_element_type=jnp.float32)
                 + b2_ref[t, k:k + 1])
        prod = x if prod is None else prod * x

    keys_ref[...] = prod / (jnp.sqrt(jnp.sum(prod * prod, axis=1,
                                             keepdims=True)) + 1e-6)


def _towers(temb, fcw, fcb, w1, b1, dw, dwb, w2, b2, bng, bnb):
    return pl.pallas_call(
        _towers_kernel,
        out_shape=jax.ShapeDtypeStruct((ROWS, C), jnp.float32),
    )(temb, fcw, fcb, w1, b1, dw, dwb, w2, b2, bng, bnb)


def _knn_kernel(v_ref, k_ref, tr_ref):
    v = v_ref[0]
    keys = k_ref[0]
    pn = v / (jnp.sqrt(jnp.sum(v * v, axis=1, keepdims=True)) + 1e-6)
    kn2 = jnp.sum(keys * keys, axis=1, keepdims=True)
    keys_aug = jnp.concatenate([keys, kn2], axis=1)
    pn_aug = jnp.concatenate(
        [pn * -2.0, jnp.ones((pn.shape[0], 1), jnp.float32)], axis=1)
    d2 = lax.dot_general(pn_aug, keys_aug, (((1,), (1,)), ((), ())),
                         preferred_element_type=jnp.float32)
    ji = lax.broadcasted_iota(jnp.int32, d2.shape, 1)
    d2 = jnp.where(ji < KEYS, d2, 3.0e38)
    m = jnp.min(d2, axis=1, keepdims=True)
    idx = jnp.min(jnp.where(d2 == m, ji, KEYSP), axis=1, keepdims=True)
    onehot = (ji == idx).astype(jnp.float32)
    tr_ref[0] = lax.dot_general(onehot, keys, (((1,), (0,)), ((), ())),
                                preferred_element_type=jnp.float32)


def _knn(vn, keys):
    return pl.pallas_call(
        _knn_kernel,
        grid=(BATCH,),
        in_specs=[
            pl.BlockSpec((1, NPIX, C), lambda b: (b, 0, 0)),
            pl.BlockSpec((1, KEYSP, C), lambda b: (b, 0, 0)),
        ],
        out_specs=pl.BlockSpec((1, NPIX, C), lambda b: (b, 0, 0)),
        out_shape=jax.ShapeDtypeStruct((BATCH, NPIX, C), jnp.float32),
    )(vn, keys)


def _scanfin_kernel(v_ref, t_ref, tvw1_ref, tvb1_ref, tvw2_ref, tvb2_ref,
                    ttw1_ref, ttb1_ref, ttw2_ref, ttb2_ref, o_ref):
    vf = v_ref[0]
    tf = t_ref[0]
    s = lax.broadcasted_iota(jnp.int32, (NPIX, 1), 0)
    hpos = s // HW
    wpos = s % HW

    def blend_coef(vcur, stride, pos):
        tprev = _shift_rows(tf, -stride)
        num = jnp.sum(vcur * tprev, axis=1, keepdims=True)
        den = jnp.maximum(
            jnp.sqrt(jnp.sum(vcur * vcur, axis=1, keepdims=True))
            * jnp.sqrt(jnp.sum(tprev * tprev, axis=1, keepdims=True)), 1e-8)
        return jnp.where(pos == 0, 0.0, jnp.exp(-(1.0 - num / den)))

    def linscan(vcur, stride, pos):
        A = blend_coef(vcur, stride, pos)
        Bv = (1.0 - A) * vcur
        k = 1
        while k < HW:
            live = pos >= k
            Am = jnp.where(live, A, 0.0)
            Bv = Am * _shift_rows(Bv, -k * stride) + Bv
            A = A * jnp.where(live, _shift_rows(A, -k * stride), 1.0)
            k *= 2
        return Bv

    vr = linscan(vf, 1, wpos)
    vc = linscan(vr, HW, hpos)

    def cosd(a, b):
        num = jnp.sum(a * b, axis=1, keepdims=True)
        den = jnp.maximum(
            jnp.sqrt(jnp.sum(a * a, axis=1, keepdims=True))
            * jnp.sqrt(jnp.sum(b * b, axis=1, keepdims=True)), 1e-8)
        return 1.0 - num / den

    d_tv = cosd(vc, tf)
    tnext = _shift_rows(tf, 1)
    d_tt = jnp.where(s == NPIX - 1, 0.0, cosd(tf, tnext))

    def mlp(d, w1, b1, w2, b2):
        h = jax.nn.relu(d * w1 + b1)
        return jnp.sum(h * w2, axis=1, keepdims=True) + b2

    gate = jax.nn.sigmoid(
        mlp(d_tv, tvw1_ref[...], tvb1_ref[...], tvw2_ref[...], tvb2_ref[...])
        + mlp(d_tt, ttw1_ref[...], ttb1_ref[...], ttw2_ref[...], ttb2_ref[...]))
    o_ref[0] = vc * gate


def _scanfin(vn, tr, mlp_params):
    vec = lambda: pl.BlockSpec((1, 256), lambda b: (0, 0))
    scl = lambda: pl.BlockSpec((1, 1), lambda b: (0, 0))
    return pl.pallas_call(
        _scanfin_kernel,
        grid=(BATCH,),
        in_specs=[
            pl.BlockSpec((1, NPIX, C), lambda b: (b, 0, 0)),
            pl.BlockSpec((1, NPIX, C), lambda b: (b, 0, 0)),
            vec(), vec(), vec(), scl(), vec(), vec(), vec(), scl(),
        ],
        out_specs=pl.BlockSpec((1, NPIX, C), lambda b: (b, 0, 0)),
        out_shape=jax.ShapeDtypeStruct((BATCH, NPIX, C), jnp.float32),
    )(vn, tr, *mlp_params)


def kernel(V, tA, tB, tAB, params):
    towers = [params[n] for n in ('tA', 'tB', 'tAB')]
    temb = jnp.stack([tA, tB, tAB])
    fcw = jnp.stack([p['fc_w'] for p in towers])
    fcb = jnp.stack([p['fc_b'] for p in towers])

    def blk(name):
        return jnp.stack([jnp.stack([b[name] for b in p['blocks']])
                          for p in towers])

    w1, b1, dwb = blk('w1'), blk('b1'), blk('dwb')
    w2, b2 = blk('w2'), blk('b2')
    bng, bnb = blk('bn_g'), blk('bn_b')
    dw = blk('dw').reshape(NT, NB, HID, 9).transpose(0, 1, 3, 2)

    keys = _towers(temb, fcw, fcb, w1, b1, dw, dwb, w2, b2, bng, bnb)
    keys = jnp.pad(keys.reshape(BATCH, KEYS, C),
                   ((0, 0), (0, KEYSP - KEYS), (0, 0)))

    vn = jnp.transpose(V, (0, 2, 3, 1)).reshape(BATCH, NPIX, C)
    tr = _knn(vn, keys)

    mlp_params = (
        params['tv']['w1'].reshape(1, 256), params['tv']['b1'].reshape(1, 256),
        params['tv']['w2'].reshape(1, 256), params['tv']['b2'].reshape(1, 1),
        params['tt']['w1'].reshape(1, 256), params['tt']['b1'].reshape(1, 256),
        params['tt']['w2'].reshape(1, 256), params['tt']['b2'].reshape(1, 1),
    )
    out = _scanfin(vn, tr, mlp_params)
    return jnp.transpose(out.reshape(BATCH, HW, HW, C), (0, 3, 1, 2))


# speedup vs baseline: 1.9487x; 1.0656x over previous
import jax
import jax.numpy as jnp
from jax import lax
from jax.experimental import pallas as pl

R = 10
HW = 64
NPIX = HW * HW
C = 128
HID = 512
NB = 4
NT = 3
BATCH = 2
ROWS = BATCH * R * R
KEYS = R * R
KEYSP = 104
NORM = float(BATCH * NPIX)


def _shift_rows(x, off):
    if off == 0:
        return x
    z = jnp.zeros((abs(off), x.shape[1]), x.dtype)
    if off > 0:
        return jnp.concatenate([x[off:], z], axis=0)
    return jnp.concatenate([z, x[:off]], axis=0)


def _towers_kernel(temb_ref, fcw_ref, fcb_ref, w1_ref, b1_ref, dw_ref,
                   dwb_ref, w2_ref, b2_ref, bng_ref, bnb_ref, keys_ref):
    s = lax.broadcasted_iota(jnp.int32, (ROWS, 1), 0)
    b_id = s // (R * R)
    hpos = (s // R) % R
    wpos = s % R
    sl = lax.broadcasted_iota(jnp.int32, (1, ROWS), 1)
    hl = (sl // R) % R
    wl = sl % R
    wt_l = (jnp.where((hl == 4) | (hl == 5), 28.0, 1.0)
            * jnp.where(wl == 7, 55.0, 1.0))
    hiprec = jax.lax.Precision.HIGHEST

    prod = None
    for t in range(NT):
        e = jnp.mean(temb_ref[t], axis=1)
        x0 = jax.nn.relu(
            lax.dot_general(e, fcw_ref[t], (((1,), (1,)), ((), ())),
                            preferred_element_type=jnp.float32)
            + fcb_ref[t:t + 1])
        x = jnp.where(b_id == 0, x0[0:1], x0[1:2])

        for k in range(NB):
            mu = lax.dot_general(wt_l, x, (((1,), (0,)), ((), ())),
                                 preferred_element_type=jnp.float32,
                                 precision=hiprec) / NORM
            ex2 = lax.dot_general(wt_l, x * x, (((1,), (0,)), ((), ())),
                                  preferred_element_type=jnp.float32,
                                  precision=hiprec) / NORM
            var = ex2 - mu * mu
            xn = (x - mu) / jnp.sqrt(var + 1e-5)
            xn = xn * bng_ref[t, k:k + 1] + bnb_ref[t, k:k + 1]
            h = jax.nn.relu(
                lax.dot_general(xn, w1_ref[t, k], (((1,), (1,)), ((), ())),
                                preferred_element_type=jnp.float32)
                + b1_ref[t, k:k + 1])
            h = jnp.where(wpos == 0, _shift_rows(h, R - 1), _shift_rows(h, -1))
            acc = None
            for ky in range(3):
                for kx in range(3):
                    dy, dx = ky - 1, kx - 1
                    m = (((hpos + dy) >= 0) & ((hpos + dy) < R)
                         & ((wpos + dx) >= 0) & ((wpos + dx) < R)
                         ).astype(h.dtype)
                    kv = dw_ref[t, k, 3 * ky + kx:3 * ky + kx + 1]
                    term = _shift_rows(h, dy * R + dx) * m * kv
                    acc = term if acc is None else acc + term
            h = jax.nn.relu(acc + dwb_ref[t, k:k + 1])
            x = (x
                 + lax.dot_general(h, w2_ref[t, k], (((1,), (1,)), ((), ())),
                                   preferred_element_type=jnp.float32)
                 + b2_ref[t, k:k + 1])
        prod = x if prod is None else prod * x

    keys_ref[...] = prod / (jnp.sqrt(jnp.sum(prod * prod, axis=1,
                                             keepdims=True)) + 1e-6)


def _towers(temb, fcw, fcb, w1, b1, dw, dwb, w2, b2, bng, bnb):
    return pl.pallas_call(
        _towers_kernel,
        out_shape=jax.ShapeDtypeStruct((ROWS, C), jnp.float32),
    )(temb, fcw, fcb, w1, b1, dw, dwb, w2, b2, bng, bnb)


def _knn_kernel(v_ref, k_ref, tr_ref):
    v = v_ref[0]
    keys = k_ref[0]
    pn = v / (jnp.sqrt(jnp.sum(v * v, axis=1, keepdims=True)) + 1e-6)
    kn2 = jnp.sum(keys * keys, axis=1, keepdims=True)
    keys_aug = jnp.concatenate([keys, kn2], axis=1)
    pn_aug = jnp.concatenate(
        [pn * -2.0, jnp.ones((pn.shape[0], 1), jnp.float32)], axis=1)
    d2 = lax.dot_general(pn_aug, keys_aug, (((1,), (1,)), ((), ())),
                         preferred_element_type=jnp.float32)
    ji = lax.broadcasted_iota(jnp.int32, d2.shape, 1)
    d2 = jnp.where(ji < KEYS, d2, 3.0e38)
    m = jnp.min(d2, axis=1, keepdims=True)
    idx = jnp.min(jnp.where(d2 == m, ji, KEYSP), axis=1, keepdims=True)
    onehot = (ji == idx).astype(jnp.float32)
    tr_ref[0] = lax.dot_general(onehot, keys, (((1,), (0,)), ((), ())),
                                preferred_element_type=jnp.float32)


def _knn(vn, keys):
    return pl.pallas_call(
        _knn_kernel,
        grid=(BATCH,),
        in_specs=[
            pl.BlockSpec((1, NPIX, C), lambda b: (b, 0, 0)),
            pl.BlockSpec((1, KEYSP, C), lambda b: (b, 0, 0)),
        ],
        out_specs=pl.BlockSpec((1, NPIX, C), lambda b: (b, 0, 0)),
        out_shape=jax.ShapeDtypeStruct((BATCH, NPIX, C), jnp.float32),
    )(vn, keys)


def _scanfin_kernel(v_ref, k_ref, tvw1_ref, tvb1_ref, tvw2_ref, tvb2_ref,
                    ttw1_ref, ttb1_ref, ttw2_ref, ttb2_ref, o_ref):
    vf = v_ref[0]
    keys = k_ref[0]
    pn = vf / (jnp.sqrt(jnp.sum(vf * vf, axis=1, keepdims=True)) + 1e-6)
    kn2 = jnp.sum(keys * keys, axis=1, keepdims=True)
    keys_aug = jnp.concatenate([keys, kn2], axis=1)
    pn_aug = jnp.concatenate(
        [pn * -2.0, jnp.ones((NPIX, 1), jnp.float32)], axis=1)
    d2 = lax.dot_general(pn_aug, keys_aug, (((1,), (1,)), ((), ())),
                         preferred_element_type=jnp.float32)
    ji = lax.broadcasted_iota(jnp.int32, d2.shape, 1)
    d2 = jnp.where(ji < KEYS, d2, 3.0e38)
    m = jnp.min(d2, axis=1, keepdims=True)
    idx = jnp.min(jnp.where(d2 == m, ji, KEYSP), axis=1, keepdims=True)
    onehot = (ji == idx).astype(jnp.float32)
    tf = lax.dot_general(onehot, keys, (((1,), (0,)), ((), ())),
                         preferred_element_type=jnp.float32)
    s = lax.broadcasted_iota(jnp.int32, (NPIX, 1), 0)
    hpos = s // HW
    wpos = s % HW

    def blend_coef(vcur, stride, pos):
        tprev = _shift_rows(tf, -stride)
        num = jnp.sum(vcur * tprev, axis=1, keepdims=True)
        den = jnp.maximum(
            jnp.sqrt(jnp.sum(vcur * vcur, axis=1, keepdims=True))
            * jnp.sqrt(jnp.sum(tprev * tprev, axis=1, keepdims=True)), 1e-8)
        return jnp.where(pos == 0, 0.0, jnp.exp(-(1.0 - num / den)))

    def linscan(vcur, stride, pos):
        A = blend_coef(vcur, stride, pos)
        Bv = (1.0 - A) * vcur
        k = 1
        while k < HW:
            live = pos >= k
            Am = jnp.where(live, A, 0.0)
            Bv = Am * _shift_rows(Bv, -k * stride) + Bv
            A = A * jnp.where(live, _shift_rows(A, -k * stride), 1.0)
            k *= 2
        return Bv

    vr = linscan(vf, 1, wpos)
    vc = linscan(vr, HW, hpos)

    def cosd(a, b):
        num = jnp.sum(a * b, axis=1, keepdims=True)
        den = jnp.maximum(
            jnp.sqrt(jnp.sum(a * a, axis=1, keepdims=True))
            * jnp.sqrt(jnp.sum(b * b, axis=1, keepdims=True)), 1e-8)
        return 1.0 - num / den

    d_tv = cosd(vc, tf)
    tnext = _shift_rows(tf, 1)
    d_tt = jnp.where(s == NPIX - 1, 0.0, cosd(tf, tnext))

    def mlp(d, w1, b1, w2, b2):
        h = jax.nn.relu(d * w1 + b1)
        return jnp.sum(h * w2, axis=1, keepdims=True) + b2

    gate = jax.nn.sigmoid(
        mlp(d_tv, tvw1_ref[...], tvb1_ref[...], tvw2_ref[...], tvb2_ref[...])
        + mlp(d_tt, ttw1_ref[...], ttb1_ref[...], ttw2_ref[...], ttb2_ref[...]))
    o_ref[0] = vc * gate


def _scanfin(vn, keys, mlp_params):
    vec = lambda: pl.BlockSpec((1, 256), lambda b: (0, 0))
    scl = lambda: pl.BlockSpec((1, 1), lambda b: (0, 0))
    return pl.pallas_call(
        _scanfin_kernel,
        grid=(BATCH,),
        in_specs=[
            pl.BlockSpec((1, NPIX, C), lambda b: (b, 0, 0)),
            pl.BlockSpec((1, KEYSP, C), lambda b: (b, 0, 0)),
            vec(), vec(), vec(), scl(), vec(), vec(), vec(), scl(),
        ],
        out_specs=pl.BlockSpec((1, NPIX, C), lambda b: (b, 0, 0)),
        out_shape=jax.ShapeDtypeStruct((BATCH, NPIX, C), jnp.float32),
    )(vn, keys, *mlp_params)


def kernel(V, tA, tB, tAB, params):
    towers = [params[n] for n in ('tA', 'tB', 'tAB')]
    temb = jnp.stack([tA, tB, tAB])
    fcw = jnp.stack([p['fc_w'] for p in towers])
    fcb = jnp.stack([p['fc_b'] for p in towers])

    def blk(name):
        return jnp.stack([jnp.stack([b[name] for b in p['blocks']])
                          for p in towers])

    w1, b1, dwb = blk('w1'), blk('b1'), blk('dwb')
    w2, b2 = blk('w2'), blk('b2')
    bng, bnb = blk('bn_g'), blk('bn_b')
    dw = blk('dw').reshape(NT, NB, HID, 9).transpose(0, 1, 3, 2)

    keys = _towers(temb, fcw, fcb, w1, b1, dw, dwb, w2, b2, bng, bnb)
    keys = jnp.pad(keys.reshape(BATCH, KEYS, C),
                   ((0, 0), (0, KEYSP - KEYS), (0, 0)))

    vn = jnp.transpose(V, (0, 2, 3, 1)).reshape(BATCH, NPIX, C)

    mlp_params = (
        params['tv']['w1'].reshape(1, 256), params['tv']['b1'].reshape(1, 256),
        params['tv']['w2'].reshape(1, 256), params['tv']['b2'].reshape(1, 1),
        params['tt']['w1'].reshape(1, 256), params['tt']['b1'].reshape(1, 256),
        params['tt']['w2'].reshape(1, 256), params['tt']['b2'].reshape(1, 1),
    )
    out = _scanfin(vn, keys, mlp_params)
    return jnp.transpose(out.reshape(BATCH, HW, HW, C), (0, 3, 1, 2))


# speedup vs baseline: 1.9531x; 1.0022x over previous
import jax
import jax.numpy as jnp
from jax import lax
from jax.experimental import pallas as pl

R = 10
HW = 64
NPIX = HW * HW
C = 128
HID = 512
NB = 4
NT = 3
BATCH = 2
ROWS = BATCH * R * R
KEYS = R * R
KEYSP = 104
NORM = float(BATCH * NPIX)


def _shift_rows(x, off):
    if off == 0:
        return x
    z = jnp.zeros((abs(off), x.shape[1]), x.dtype)
    if off > 0:
        return jnp.concatenate([x[off:], z], axis=0)
    return jnp.concatenate([z, x[:off]], axis=0)


def _towers_kernel(temb_ref, fcw_ref, fcb_ref, w1_ref, b1_ref, dw_ref,
                   dwb_ref, w2_ref, b2_ref, bng_ref, bnb_ref, keys_ref):
    s = lax.broadcasted_iota(jnp.int32, (ROWS, 1), 0)
    b_id = s // (R * R)
    hpos = (s // R) % R
    wpos = s % R
    sl = lax.broadcasted_iota(jnp.int32, (1, ROWS), 1)
    hl = (sl // R) % R
    wl = sl % R
    wt_l = (jnp.where((hl == 4) | (hl == 5), 28.0, 1.0)
            * jnp.where(wl == 7, 55.0, 1.0))
    hiprec = jax.lax.Precision.HIGHEST

    prod = None
    for t in range(NT):
        e = jnp.mean(temb_ref[t], axis=1)
        x0 = jax.nn.relu(
            lax.dot_general(e, fcw_ref[t], (((1,), (1,)), ((), ())),
                            preferred_element_type=jnp.float32)
            + fcb_ref[t:t + 1])
        x = jnp.where(b_id == 0, x0[0:1], x0[1:2])

        for k in range(NB):
            mu = lax.dot_general(wt_l, x, (((1,), (0,)), ((), ())),
                                 preferred_element_type=jnp.float32,
                                 precision=hiprec) / NORM
            ex2 = lax.dot_general(wt_l, x * x, (((1,), (0,)), ((), ())),
                                  preferred_element_type=jnp.float32,
                                  precision=hiprec) / NORM
            var = ex2 - mu * mu
            xn = (x - mu) / jnp.sqrt(var + 1e-5)
            xn = xn * bng_ref[t, k:k + 1] + bnb_ref[t, k:k + 1]
            h = jax.nn.relu(
                lax.dot_general(xn, w1_ref[t, k], (((1,), (1,)), ((), ())),
                                preferred_element_type=jnp.float32)
                + b1_ref[t, k:k + 1])
            h = jnp.where(wpos == 0, _shift_rows(h, R - 1), _shift_rows(h, -1))
            acc = None
            for ky in range(3):
                for kx in range(3):
                    dy, dx = ky - 1, kx - 1
                    m = (((hpos + dy) >= 0) & ((hpos + dy) < R)
                         & ((wpos + dx) >= 0) & ((wpos + dx) < R)
                         ).astype(h.dtype)
                    kv = dw_ref[t, k, 3 * ky + kx:3 * ky + kx + 1]
                    term = _shift_rows(h, dy * R + dx) * m * kv
                    acc = term if acc is None else acc + term
            h = jax.nn.relu(acc + dwb_ref[t, k:k + 1])
            x = (x
                 + lax.dot_general(h, w2_ref[t, k], (((1,), (1,)), ((), ())),
                                   preferred_element_type=jnp.float32)
                 + b2_ref[t, k:k + 1])
        prod = x if prod is None else prod * x

    keys_ref[...] = prod / (jnp.sqrt(jnp.sum(prod * prod, axis=1,
                                             keepdims=True)) + 1e-6)


def _towers(temb, fcw, fcb, w1, b1, dw, dwb, w2, b2, bng, bnb):
    return pl.pallas_call(
        _towers_kernel,
        out_shape=jax.ShapeDtypeStruct((ROWS, C), jnp.float32),
    )(temb, fcw, fcb, w1, b1, dw, dwb, w2, b2, bng, bnb)


def _scanfin_kernel(v_ref, k_ref, tvw1_ref, tvb1_ref, tvw2_ref, tvb2_ref,
                    ttw1_ref, ttb1_ref, ttw2_ref, ttb2_ref, o_ref):
    vf = v_ref[0]
    keys = k_ref[0]
    pn = vf / (jnp.sqrt(jnp.sum(vf * vf, axis=1, keepdims=True)) + 1e-6)
    kn2 = jnp.sum(keys * keys, axis=1, keepdims=True)
    keys_aug = jnp.concatenate([keys, kn2], axis=1)
    pn_aug = jnp.concatenate(
        [pn * -2.0, jnp.ones((NPIX, 1), jnp.float32)], axis=1)
    d2 = lax.dot_general(pn_aug, keys_aug, (((1,), (1,)), ((), ())),
                         preferred_element_type=jnp.float32)
    ji = lax.broadcasted_iota(jnp.int32, d2.shape, 1)
    d2 = jnp.where(ji < KEYS, d2, 3.0e38)
    m = jnp.min(d2, axis=1, keepdims=True)
    idx = jnp.min(jnp.where(d2 == m, ji, KEYSP), axis=1, keepdims=True)
    onehot = (ji == idx).astype(jnp.float32)
    tf = lax.dot_general(onehot, keys, (((1,), (0,)), ((), ())),
                         preferred_element_type=jnp.float32)
    s = lax.broadcasted_iota(jnp.int32, (NPIX, 1), 0)
    hpos = s // HW
    wpos = s % HW

    def blend_coef(vcur, stride, pos):
        tprev = _shift_rows(tf, -stride)
        num = jnp.sum(vcur * tprev, axis=1, keepdims=True)
        den = jnp.maximum(
            jnp.sqrt(jnp.sum(vcur * vcur, axis=1, keepdims=True))
            * jnp.sqrt(jnp.sum(tprev * tprev, axis=1, keepdims=True)), 1e-8)
        return jnp.where(pos == 0, 0.0, jnp.exp(-(1.0 - num / den)))

    def linscan(vcur, stride, pos):
        A = blend_coef(vcur, stride, pos)
        Bv = (1.0 - A) * vcur
        k = 1
        while k < HW:
            live = pos >= k
            Am = jnp.where(live, A, 0.0)
            Bv = Am * _shift_rows(Bv, -k * stride) + Bv
            A = A * jnp.where(live, _shift_rows(A, -k * stride), 1.0)
            k *= 2
        return Bv

    vr = linscan(vf, 1, wpos)
    vc = linscan(vr, HW, hpos)

    def cosd(a, b):
        num = jnp.sum(a * b, axis=1, keepdims=True)
        den = jnp.maximum(
            jnp.sqrt(jnp.sum(a * a, axis=1, keepdims=True))
            * jnp.sqrt(jnp.sum(b * b, axis=1, keepdims=True)), 1e-8)
        return 1.0 - num / den

    d_tv = cosd(vc, tf)
    tnext = _shift_rows(tf, 1)
    d_tt = jnp.where(s == NPIX - 1, 0.0, cosd(tf, tnext))

    def mlp(d, w1, b1, w2, b2):
        h = jax.nn.relu(d * w1 + b1)
        return jnp.sum(h * w2, axis=1, keepdims=True) + b2

    gate = jax.nn.sigmoid(
        mlp(d_tv, tvw1_ref[...], tvb1_ref[...], tvw2_ref[...], tvb2_ref[...])
        + mlp(d_tt, ttw1_ref[...], ttb1_ref[...], ttw2_ref[...], ttb2_ref[...]))
    o_ref[0] = vc * gate


def _scanfin(vn, keys, mlp_params):
    vec = lambda: pl.BlockSpec((1, 256), lambda b: (0, 0))
    scl = lambda: pl.BlockSpec((1, 1), lambda b: (0, 0))
    return pl.pallas_call(
        _scanfin_kernel,
        grid=(BATCH,),
        in_specs=[
            pl.BlockSpec((1, NPIX, C), lambda b: (b, 0, 0)),
            pl.BlockSpec((1, KEYSP, C), lambda b: (b, 0, 0)),
            vec(), vec(), vec(), scl(), vec(), vec(), vec(), scl(),
        ],
        out_specs=pl.BlockSpec((1, NPIX, C), lambda b: (b, 0, 0)),
        out_shape=jax.ShapeDtypeStruct((BATCH, NPIX, C), jnp.float32),
    )(vn, keys, *mlp_params)


def kernel(V, tA, tB, tAB, params):
    towers = [params[n] for n in ('tA', 'tB', 'tAB')]
    temb = jnp.stack([tA, tB, tAB])
    fcw = jnp.stack([p['fc_w'] for p in towers])
    fcb = jnp.stack([p['fc_b'] for p in towers])

    def blk(name):
        return jnp.stack([jnp.stack([b[name] for b in p['blocks']])
                          for p in towers])

    w1, b1, dwb = blk('w1'), blk('b1'), blk('dwb')
    w2, b2 = blk('w2'), blk('b2')
    bng, bnb = blk('bn_g'), blk('bn_b')
    dw = blk('dw').reshape(NT, NB, HID, 9).transpose(0, 1, 3, 2)

    keys = _towers(temb, fcw, fcb, w1, b1, dw, dwb, w2, b2, bng, bnb)
    keys = jnp.pad(keys.reshape(BATCH, KEYS, C),
                   ((0, 0), (0, KEYSP - KEYS), (0, 0)))

    vn = jnp.transpose(V, (0, 2, 3, 1)).reshape(BATCH, NPIX, C)

    mlp_params = (
        params['tv']['w1'].reshape(1, 256), params['tv']['b1'].reshape(1, 256),
        params['tv']['w2'].reshape(1, 256), params['tv']['b2'].reshape(1, 1),
        params['tt']['w1'].reshape(1, 256), params['tt']['b1'].reshape(1, 256),
        params['tt']['w2'].reshape(1, 256), params['tt']['b2'].reshape(1, 1),
    )
    out = _scanfin(vn, keys, mlp_params)
    return jnp.transpose(out.reshape(BATCH, HW, HW, C), (0, 3, 1, 2))
